# baseline (speedup 1.0000x reference)
import functools

import jax
import jax.numpy as jnp
from jax import lax
from jax.experimental import pallas as pl
from jax.experimental.pallas import tpu as pltpu
from jax.experimental.pallas import tpu_sc as plsc

B, N, M, C1, C2 = 8, 4096, 1024, 64, 128
TN = 512
BN = B * N
NW = 32
PW = BN // NW
CP = 64
NCHUNK = PW // CP

_BIG = 1e30


def _three_nn_kernel(b0, unknown_ref, known_ref,
                     i0_ref, i1_ref, i2_ref, w0_ref, w1_ref, w2_ref):
    b = pl.program_id(0) + b0
    u = unknown_ref[0]
    k = known_ref[0]

    ut = jnp.transpose(-2.0 * u)
    u2 = 0.25 * jnp.sum(ut * ut, axis=0, keepdims=True)
    k2 = jnp.sum(k * k, axis=-1, keepdims=True)
    cross = jax.lax.dot_general(
        k, ut, (((1,), (0,)), ((), ())),
        preferred_element_type=jnp.float32)
    d2 = (u2 + cross) + k2

    iota = jax.lax.broadcasted_iota(jnp.int32, (M, TN), 0)

    def pick(d):
        m = jnp.min(d, axis=0, keepdims=True)
        eq = d == m
        idx = jnp.min(jnp.where(eq, iota, M), axis=0, keepdims=True)
        return m, idx, jnp.where(eq, _BIG, d)

    m0, i0, d2a = pick(d2)
    m1, i1, d2b = pick(d2a)
    m2, i2, _ = pick(d2b)

    r0 = 1.0 / (jnp.maximum(m0, 0.0) + 1e-8)
    r1 = 1.0 / (jnp.maximum(m1, 0.0) + 1e-8)
    r2 = 1.0 / (jnp.maximum(m2, 0.0) + 1e-8)
    inv_norm = 1.0 / (r0 + r1 + r2)

    base = b * M
    i0_ref[:] = (i0 + base)[0, :]
    i1_ref[:] = (i1 + base)[0, :]
    i2_ref[:] = (i2 + base)[0, :]
    w0_ref[:] = (r0 * inv_norm)[0, :]
    w1_ref[:] = (r1 * inv_norm)[0, :]
    w2_ref[:] = (r2 * inv_norm)[0, :]


def _three_nn(unknown, known, b0, nb):
    grid = (nb, N // TN)
    npts = nb * N
    flat_spec = pl.BlockSpec((TN,), lambda b, i: (b * (N // TN) + i,))
    return pl.pallas_call(
        functools.partial(_three_nn_kernel, b0),
        grid=grid,
        in_specs=[
            pl.BlockSpec((1, TN, 3), lambda b, i: (b, i, 0)),
            pl.BlockSpec((1, M, 3), lambda b, i: (b, 0, 0)),
        ],
        out_specs=[flat_spec] * 6,
        out_shape=[jax.ShapeDtypeStruct((npts,), jnp.int32)] * 3
        + [jax.ShapeDtypeStruct((npts,), jnp.float32)] * 3,
    )(unknown, known)


_GDN = lax.GatherDimensionNumbers(
    offset_dims=(), collapsed_slice_dims=(0,), start_index_map=(0,))


def _bcast(vec16, j):
    jv = jnp.full((16, 1), j, jnp.int32)
    return lax.gather(vec16, jv, _GDN, slice_sizes=(1,),
                      mode=lax.GatherScatterMode.PROMISE_IN_BOUNDS)


@functools.cache
def _build_sc_interp(npts, base0):
    pw = npts // NW
    nchunk = pw // CP
    mesh = plsc.VectorSubcoreMesh(core_axis_name="c", subcore_axis_name="s")

    @functools.partial(
        pl.kernel,
        mesh=mesh,
        out_type=jax.ShapeDtypeStruct((npts, C1 + C2), jnp.float32),
        scratch_types=[pltpu.VMEM((CP,), jnp.int32)] * 6
        + [pltpu.VMEM((CP,), jnp.float32)] * 6
        + [pltpu.VMEM((CP, C2), jnp.float32)] * 6
        + [pltpu.VMEM((CP, C1), jnp.float32)] * 2
        + [pltpu.SemaphoreType.DMA] * 2,
    )
    def _sc_interp(table_hbm, i0_hbm, i1_hbm, i2_hbm, w0_hbm, w1_hbm, w2_hbm,
                   uf_hbm, out_hbm,
                   i0a, i1a, i2a, i0b, i1b, i2b,
                   w0a, w1a, w2a, w0b, w1b, w2b,
                   g0a, g1a, g2a, g0b, g1b, g2b,
                   ufa, ufb, sema, semb):
        wid = lax.axis_index("s") * 2 + lax.axis_index("c")
        wbase = wid * pw
        sets = [
            (i0a, i1a, i2a, w0a, w1a, w2a, g0a, g1a, g2a, ufa, sema),
            (i0b, i1b, i2b, w0b, w1b, w2b, g0b, g1b, g2b, ufb, semb),
        ]
        handles = {}

        def load(t):
            i0v, i1v, i2v, w0v, w1v, w2v, g0v, g1v, g2v, ufv, sem = sets[t % 2]
            base = wbase + t * CP
            pltpu.sync_copy(i0_hbm.at[pl.ds(base, CP)], i0v)
            pltpu.sync_copy(i1_hbm.at[pl.ds(base, CP)], i1v)
            pltpu.sync_copy(i2_hbm.at[pl.ds(base, CP)], i2v)
            pltpu.sync_copy(w0_hbm.at[pl.ds(base, CP)], w0v)
            pltpu.sync_copy(w1_hbm.at[pl.ds(base, CP)], w1v)
            pltpu.sync_copy(w2_hbm.at[pl.ds(base, CP)], w2v)
            pltpu.sync_copy(uf_hbm.at[pl.ds(base0 + base, CP), :], ufv)
            handles[t] = (pltpu.async_copy(table_hbm.at[i0v], g0v, sem),
                          pltpu.async_copy(table_hbm.at[i1v], g1v, sem),
                          pltpu.async_copy(table_hbm.at[i2v], g2v, sem))

        load(0)
        for t in range(nchunk):
            if t + 1 < nchunk:
                load(t + 1)
            for a in handles[t]:
                a.wait()
            _, _, _, w0v, w1v, w2v, g0v, g1v, g2v, ufv, _ = sets[t % 2]
            base = wbase + t * CP

            def group_body(q, carry2, w0v=w0v, w1v=w1v, w2v=w2v,
                           g0v=g0v, g1v=g1v, g2v=g2v):
                wq0 = w0v[pl.ds(q * 16, 16)]
                wq1 = w1v[pl.ds(q * 16, 16)]
                wq2 = w2v[pl.ds(q * 16, 16)]
                for j in range(16):
                    p = q * 16 + j
                    w0 = _bcast(wq0, j)
                    w1 = _bcast(wq1, j)
                    w2 = _bcast(wq2, j)
                    for f in range(C2 // 16):
                        sl = pl.ds(f * 16, 16)
                        g0v[p, sl] = (w0 * g0v[p, sl] + w1 * g1v[p, sl]
                                      + w2 * g2v[p, sl])
                return carry2

            lax.fori_loop(0, CP // 16, group_body, 0)
            pltpu.sync_copy(g0v, out_hbm.at[pl.ds(base, CP), pl.ds(0, C2)])
            pltpu.sync_copy(ufv, out_hbm.at[pl.ds(base, CP), pl.ds(C2, C1)])

    return _sc_interp


NSPLIT = 2
HB = B // NSPLIT
HP = HB * N


@jax.jit
def kernel(unknown, known, unknow_feats, known_feats):
    table = known_feats.reshape(B * M, C2)
    uf_flat = unknow_feats.reshape(BN, C1)
    parts = []
    for h in range(NSPLIT):
        bs = slice(h * HB, (h + 1) * HB)
        i0, i1, i2, w0, w1, w2 = _three_nn(unknown[bs], known[bs], h * HB, HB)
        parts.append(_build_sc_interp(HP, h * HP)(
            table, i0, i1, i2, w0, w1, w2, uf_flat))
    out = jnp.concatenate(parts, axis=0).reshape(B, N, C1 + C2)
    return (out, out)
